# baseline (speedup 1.0000x reference)
import functools

import jax
import jax.numpy as jnp
from jax import lax
from jax.experimental import pallas as pl
from jax.experimental.pallas import tpu as pltpu
from jax.experimental.pallas import tpu_sc as plsc

_NC = 2
_NS = 16
_NW = _NC * _NS
_L = 16


def _sc_energy(n_pairs, n_atoms, chunk):
    n_per_w = n_pairs // _NW
    n_chunks = n_per_w // chunk
    n_steps = chunk // _L
    c2 = 2 * chunk

    mesh = plsc.VectorSubcoreMesh(core_axis_name="c", subcore_axis_name="s")

    @functools.partial(
        pl.kernel,
        mesh=mesh,
        compiler_params=pltpu.CompilerParams(
            needs_layout_passes=False, use_tc_tiling_on_sc=False),
        out_type=jax.ShapeDtypeStruct((_NW, _L), jnp.float32),
        scratch_types=[
            pltpu.VMEM((n_atoms,), jnp.int32),
            pltpu.VMEM((1024,), jnp.float32),
            pltpu.VMEM((1024,), jnp.float32),
            pltpu.VMEM((112,), jnp.float32),
            pltpu.VMEM((c2,), jnp.int32),
            pltpu.VMEM((c2,), jnp.int32),
            pltpu.VMEM((c2, 8), jnp.float32),
            pltpu.VMEM((c2, 8), jnp.float32),
            pltpu.VMEM((_L,), jnp.float32),
            pltpu.SemaphoreType.DMA,
            pltpu.SemaphoreType.DMA,
        ],
    )
    def body(coords8, pf, sig, eps, types, cst, out,
             types_v, sig_v, eps_v, cst_v, ib_a, ib_b, rb_a, rb_b, acc_v,
             sem_a, sem_b):
        wid = lax.axis_index("s") * _NC + lax.axis_index("c")
        base_w = wid * (2 * n_per_w)

        pltpu.sync_copy(types, types_v)
        pltpu.sync_copy(sig, sig_v)
        pltpu.sync_copy(eps, eps_v)
        pltpu.sync_copy(cst, cst_v)

        ibx = cst_v[pl.ds(0, _L)]
        iby = cst_v[pl.ds(16, _L)]
        ibz = cst_v[pl.ds(32, _L)]
        lxv = cst_v[pl.ds(48, _L)]
        lyv = cst_v[pl.ds(64, _L)]
        lzv = cst_v[pl.ds(80, _L)]
        cut2 = cst_v[pl.ds(96, _L)]
        lanes = lax.iota(jnp.int32, _L)
        lanes2 = lanes * 2

        def bf16r(x):
            u = plsc.bitcast(x, jnp.int32)
            u = u + 0x7FFF + ((u >> 16) & 1)
            u = u & jnp.int32(-65536)
            return plsc.bitcast(u, jnp.float32)

        def minimg(d, ib, lv):
            s = bf16r(d) * ib
            f = jnp.where(s > 0.5, 1.0, 0.0) + jnp.where(s < -0.5, -1.0, 0.0)
            return bf16r(s - f) * lv

        def fetch(g, ib, rb, sem):
            base = pl.multiple_of(base_w + g * c2, 8)
            pltpu.sync_copy(pf.at[pl.ds(base, c2)], ib)
            pltpu.async_copy(coords8.at[ib], rb, sem)

        def drain(ib, rb, sem):
            pltpu.make_async_copy(coords8.at[ib], rb, sem).wait()

        def compute(ib, rb, acc):
            def step(i, acc):
                e = lanes2 + i * 32
                iv = plsc.load_gather(ib, [e])
                jv = plsc.load_gather(ib, [e + 1])
                ti = plsc.load_gather(types_v, [iv])
                tj = plsc.load_gather(types_v, [jv])
                tp = ti * 32 + tj
                sg = plsc.load_gather(sig_v, [tp])
                ep = plsc.load_gather(eps_v, [tp])
                c0 = lanes - lanes
                xi = plsc.load_gather(rb, [e, c0])
                yi = plsc.load_gather(rb, [e, c0 + 1])
                zi = plsc.load_gather(rb, [e, c0 + 2])
                xj = plsc.load_gather(rb, [e + 1, c0])
                yj = plsc.load_gather(rb, [e + 1, c0 + 1])
                zj = plsc.load_gather(rb, [e + 1, c0 + 2])
                dx = minimg(xi - xj, ibx, lxv)
                dy = minimg(yi - yj, iby, lyv)
                dz = minimg(zi - zj, ibz, lzv)
                r2 = dx * dx + dy * dy + dz * dz
                r2 = jnp.maximum(r2, 1e-6)
                s2 = sg * sg / r2
                tmp = s2 * s2 * s2
                ene = 4.0 * ep * tmp * (tmp - 1.0)
                ene = jnp.where(r2 < cut2, ene, jnp.zeros_like(ene))
                return acc + ene

            return lax.fori_loop(0, n_steps, step, acc)

        fetch(0, ib_a, rb_a, sem_a)

        def chunk_pair(t, acc):
            fetch(2 * t + 1, ib_b, rb_b, sem_b)
            drain(ib_a, rb_a, sem_a)
            acc = compute(ib_a, rb_a, acc)

            @pl.when(t + 1 < n_chunks // 2)
            def _():
                fetch(2 * t + 2, ib_a, rb_a, sem_a)

            drain(ib_b, rb_b, sem_b)
            return compute(ib_b, rb_b, acc)

        acc = lax.fori_loop(0, n_chunks // 2, chunk_pair,
                            jnp.zeros((_L,), jnp.float32))
        acc_v[...] = acc
        pltpu.sync_copy(acc_v, out.at[wid])

    return body


def _tc_sum(x_ref, o_ref):
    o_ref[0, 0] = jnp.sum(x_ref[...])


def kernel(coords, pairs, box, sigma, epsilon, cutoff, atom_types):
    n_pairs = pairs.shape[0]
    n_atoms = coords.shape[0]
    chunk = 2000
    assert n_pairs % (_NW * chunk) == 0 and (n_pairs // (_NW * chunk)) % 2 == 0

    coords8 = jnp.pad(coords.astype(jnp.float32), ((0, 0), (0, 5)))
    pf = pairs.astype(jnp.int32).reshape(-1)
    sig = sigma.astype(jnp.float32).reshape(-1)
    eps = epsilon.astype(jnp.float32).reshape(-1)
    box = box.astype(jnp.float32)
    inv_box = jnp.linalg.inv(box)
    cut = jnp.asarray(cutoff, jnp.float32)

    def bf(x):
        return x.astype(jnp.bfloat16).astype(jnp.float32)

    cst = jnp.concatenate([
        jnp.full((16,), bf(inv_box[0, 0]), jnp.float32),
        jnp.full((16,), bf(inv_box[1, 1]), jnp.float32),
        jnp.full((16,), bf(inv_box[2, 2]), jnp.float32),
        jnp.full((16,), bf(box[0, 0]), jnp.float32),
        jnp.full((16,), bf(box[1, 1]), jnp.float32),
        jnp.full((16,), bf(box[2, 2]), jnp.float32),
        jnp.full((16,), cut * cut, jnp.float32),
    ])

    parts = _sc_energy(n_pairs, n_atoms, chunk)(
        coords8, pf, sig, eps, atom_types.astype(jnp.int32), cst)

    total = pl.pallas_call(
        _tc_sum,
        out_shape=jax.ShapeDtypeStruct((1, 1), jnp.float32),
        out_specs=pl.BlockSpec(memory_space=pltpu.SMEM),
    )(parts)
    return total[0, 0]


# speedup vs baseline: 1.0005x; 1.0005x over previous
import functools

import jax
import jax.numpy as jnp
from jax import lax
from jax.experimental import pallas as pl
from jax.experimental.pallas import tpu as pltpu
from jax.experimental.pallas import tpu_sc as plsc

_NC = 2
_NS = 16
_NW = _NC * _NS
_L = 16


def _sc_energy(n_pairs, n_atoms, chunk):
    n_per_w = n_pairs // _NW
    n_chunks = n_per_w // chunk
    n_steps = chunk // _L
    c2 = 2 * chunk

    mesh = plsc.VectorSubcoreMesh(core_axis_name="c", subcore_axis_name="s")

    @functools.partial(
        pl.kernel,
        mesh=mesh,
        compiler_params=pltpu.CompilerParams(
            needs_layout_passes=False, use_tc_tiling_on_sc=False),
        out_type=jax.ShapeDtypeStruct((_NW, _L), jnp.float32),
        scratch_types=[
            pltpu.VMEM((n_atoms,), jnp.int32),
            pltpu.VMEM((1024,), jnp.float32),
            pltpu.VMEM((1024,), jnp.float32),
            pltpu.VMEM((112,), jnp.float32),
            pltpu.VMEM((c2,), jnp.int32),
            pltpu.VMEM((c2,), jnp.int32),
            pltpu.VMEM((c2, 8), jnp.float32),
            pltpu.VMEM((c2, 8), jnp.float32),
            pltpu.VMEM((_L,), jnp.float32),
            pltpu.SemaphoreType.DMA,
            pltpu.SemaphoreType.DMA,
        ],
    )
    def body(coords8, pf, sig, eps, types, cst, out,
             types_v, sig_v, eps_v, cst_v, ib_a, ib_b, rb_a, rb_b, acc_v,
             sem_a, sem_b):
        wid = lax.axis_index("s") * _NC + lax.axis_index("c")
        base_w = wid * (2 * n_per_w)

        pltpu.sync_copy(types, types_v)
        pltpu.sync_copy(sig, sig_v)
        pltpu.sync_copy(eps, eps_v)
        pltpu.sync_copy(cst, cst_v)

        ibx = cst_v[pl.ds(0, _L)]
        iby = cst_v[pl.ds(16, _L)]
        ibz = cst_v[pl.ds(32, _L)]
        lxv = cst_v[pl.ds(48, _L)]
        lyv = cst_v[pl.ds(64, _L)]
        lzv = cst_v[pl.ds(80, _L)]
        cut2 = cst_v[pl.ds(96, _L)]
        lanes = lax.iota(jnp.int32, _L)
        lanes2 = lanes * 2

        def bf16r(x):
            u = plsc.bitcast(x, jnp.int32)
            u = u + 0x7FFF + ((u >> 16) & 1)
            u = u & jnp.int32(-65536)
            return plsc.bitcast(u, jnp.float32)

        def minimg(d, ib, lv):
            s = bf16r(d) * ib
            f = jnp.where(s > 0.5, 1.0, 0.0) + jnp.where(s < -0.5, -1.0, 0.0)
            return bf16r(s - f) * lv

        def fetch(g, ib, rb, sem):
            base = pl.multiple_of(base_w + g * c2, 8)
            pltpu.sync_copy(pf.at[pl.ds(base, c2)], ib)
            pltpu.async_copy(coords8.at[ib], rb, sem)

        def drain(ib, rb, sem):
            pltpu.make_async_copy(coords8.at[ib], rb, sem).wait()

        def compute(ib, rb, acc):
            def step(i, acc):
                e = lanes2 + i * 32
                iv = plsc.load_gather(ib, [e])
                jv = plsc.load_gather(ib, [e + 1])
                ti = plsc.load_gather(types_v, [iv])
                tj = plsc.load_gather(types_v, [jv])
                tp = ti * 32 + tj
                sg = plsc.load_gather(sig_v, [tp])
                ep = plsc.load_gather(eps_v, [tp])
                c0 = lanes - lanes
                xi = plsc.load_gather(rb, [e, c0])
                yi = plsc.load_gather(rb, [e, c0 + 1])
                zi = plsc.load_gather(rb, [e, c0 + 2])
                xj = plsc.load_gather(rb, [e + 1, c0])
                yj = plsc.load_gather(rb, [e + 1, c0 + 1])
                zj = plsc.load_gather(rb, [e + 1, c0 + 2])
                dx = minimg(xi - xj, ibx, lxv)
                dy = minimg(yi - yj, iby, lyv)
                dz = minimg(zi - zj, ibz, lzv)
                r2 = dx * dx + dy * dy + dz * dz
                r2 = jnp.maximum(r2, 1e-6)
                s2 = sg * sg / r2
                tmp = s2 * s2 * s2
                ene = 4.0 * ep * tmp * (tmp - 1.0)
                ene = jnp.where(r2 < cut2, ene, jnp.zeros_like(ene))
                return acc + ene

            return lax.fori_loop(0, n_steps, step, acc)

        fetch(0, ib_a, rb_a, sem_a)

        def chunk_pair(t, acc):
            fetch(2 * t + 1, ib_b, rb_b, sem_b)
            drain(ib_a, rb_a, sem_a)
            acc = compute(ib_a, rb_a, acc)

            @pl.when(t + 1 < n_chunks // 2)
            def _():
                fetch(2 * t + 2, ib_a, rb_a, sem_a)

            drain(ib_b, rb_b, sem_b)
            return compute(ib_b, rb_b, acc)

        acc = lax.fori_loop(0, n_chunks // 2, chunk_pair,
                            jnp.zeros((_L,), jnp.float32))
        acc_v[...] = acc
        pltpu.sync_copy(acc_v, out.at[wid])

    return body


def _tc_sum(x_ref, o_ref):
    o_ref[0, 0] = jnp.sum(x_ref[...])


def kernel(coords, pairs, box, sigma, epsilon, cutoff, atom_types):
    n_pairs = pairs.shape[0]
    n_atoms = coords.shape[0]
    chunk = 2000
    assert n_pairs % (_NW * chunk) == 0 and (n_pairs // (_NW * chunk)) % 2 == 0

    coords8 = jnp.pad(coords.astype(jnp.float32), ((0, 0), (0, 5)))
    rt0 = jnp.asarray(cutoff, jnp.int32) * 0
    pf = (pairs.astype(jnp.int32) ^ rt0).reshape(-1)
    sig = sigma.astype(jnp.float32).reshape(-1)
    eps = epsilon.astype(jnp.float32).reshape(-1)
    box = box.astype(jnp.float32)
    inv_box = jnp.linalg.inv(box)
    cut = jnp.asarray(cutoff, jnp.float32)

    def bf(x):
        return x.astype(jnp.bfloat16).astype(jnp.float32)

    cst = jnp.concatenate([
        jnp.full((16,), bf(inv_box[0, 0]), jnp.float32),
        jnp.full((16,), bf(inv_box[1, 1]), jnp.float32),
        jnp.full((16,), bf(inv_box[2, 2]), jnp.float32),
        jnp.full((16,), bf(box[0, 0]), jnp.float32),
        jnp.full((16,), bf(box[1, 1]), jnp.float32),
        jnp.full((16,), bf(box[2, 2]), jnp.float32),
        jnp.full((16,), cut * cut, jnp.float32),
    ])

    parts = _sc_energy(n_pairs, n_atoms, chunk)(
        coords8, pf, sig, eps, atom_types.astype(jnp.int32), cst)

    total = pl.pallas_call(
        _tc_sum,
        out_shape=jax.ShapeDtypeStruct((1, 1), jnp.float32),
        out_specs=pl.BlockSpec(memory_space=pltpu.SMEM),
    )(parts)
    return total[0, 0]


# speedup vs baseline: 11.8994x; 11.8934x over previous
import functools

import jax
import jax.numpy as jnp
from jax import lax
from jax.experimental import pallas as pl
from jax.experimental.pallas import tpu as pltpu
from jax.experimental.pallas import tpu_sc as plsc

_NC = 2
_NS = 16
_NW = _NC * _NS
_L = 16


def _sc_energy(n_pairs, n_atoms, chunk):
    n_per_w = n_pairs // _NW
    n_chunks = n_per_w // chunk
    n_steps = chunk // _L
    c2 = 2 * chunk

    mesh = plsc.VectorSubcoreMesh(core_axis_name="c", subcore_axis_name="s")

    @functools.partial(
        pl.kernel,
        mesh=mesh,
        compiler_params=pltpu.CompilerParams(
            needs_layout_passes=False, use_tc_tiling_on_sc=False),
        out_type=jax.ShapeDtypeStruct((_NW, _L), jnp.float32),
        scratch_types=[
            pltpu.VMEM((n_atoms,), jnp.int32),
            pltpu.VMEM((1024,), jnp.float32),
            pltpu.VMEM((1024,), jnp.float32),
            pltpu.VMEM((112,), jnp.float32),
            pltpu.VMEM((chunk,), jnp.int32),
            pltpu.VMEM((chunk,), jnp.int32),
            pltpu.VMEM((chunk,), jnp.int32),
            pltpu.VMEM((chunk,), jnp.int32),
            pltpu.VMEM((chunk, 8), jnp.float32),
            pltpu.VMEM((chunk, 8), jnp.float32),
            pltpu.VMEM((chunk, 8), jnp.float32),
            pltpu.VMEM((chunk, 8), jnp.float32),
            pltpu.VMEM((_L,), jnp.float32),
            pltpu.SemaphoreType.DMA,
            pltpu.SemaphoreType.DMA,
        ],
    )
    def body(coords8, pi, pj, sig, eps, types, cst, out,
             types_v, sig_v, eps_v, cst_v,
             ibi_a, ibj_a, ibi_b, ibj_b, ri_a, rj_a, ri_b, rj_b, acc_v,
             sem_a, sem_b):
        wid = lax.axis_index("s") * _NC + lax.axis_index("c")
        base_w = wid * n_per_w

        pltpu.sync_copy(types, types_v)
        pltpu.sync_copy(sig, sig_v)
        pltpu.sync_copy(eps, eps_v)
        pltpu.sync_copy(cst, cst_v)

        ibx = cst_v[pl.ds(0, _L)]
        iby = cst_v[pl.ds(16, _L)]
        ibz = cst_v[pl.ds(32, _L)]
        lxv = cst_v[pl.ds(48, _L)]
        lyv = cst_v[pl.ds(64, _L)]
        lzv = cst_v[pl.ds(80, _L)]
        cut2 = cst_v[pl.ds(96, _L)]
        lanes = lax.iota(jnp.int32, _L)
        lanes2 = lanes * 2

        def bf16r(x):
            u = plsc.bitcast(x, jnp.int32)
            u = u + 0x7FFF + ((u >> 16) & 1)
            u = u & jnp.int32(-65536)
            return plsc.bitcast(u, jnp.float32)

        def minimg(d, ib, lv):
            s = bf16r(d) * ib
            f = jnp.where(s > 0.5, 1.0, 0.0) + jnp.where(s < -0.5, -1.0, 0.0)
            return bf16r(s - f) * lv

        def fetch(g, ibi, ibj, ri, rj, sem):
            base = pl.multiple_of(base_w + g * chunk, 8)
            pltpu.sync_copy(pi.at[pl.ds(base, chunk)], ibi)
            pltpu.sync_copy(pj.at[pl.ds(base, chunk)], ibj)
            pltpu.async_copy(coords8.at[ibi], ri, sem)
            pltpu.async_copy(coords8.at[ibj], rj, sem)

        def drain(ibi, ibj, ri, rj, sem):
            pltpu.make_async_copy(coords8.at[ibi], ri, sem).wait()
            pltpu.make_async_copy(coords8.at[ibj], rj, sem).wait()

        def compute(ibi, ibj, ri, rj, acc):
            def step(i, acc):
                o = i * _L
                e = lanes + o
                iv = ibi[pl.ds(o, _L)]
                jv = ibj[pl.ds(o, _L)]
                ti = plsc.load_gather(types_v, [iv])
                tj = plsc.load_gather(types_v, [jv])
                tp = ti * 32 + tj
                sg = plsc.load_gather(sig_v, [tp])
                ep = plsc.load_gather(eps_v, [tp])
                c0 = lanes - lanes
                xi = plsc.load_gather(ri, [e, c0])
                yi = plsc.load_gather(ri, [e, c0 + 1])
                zi = plsc.load_gather(ri, [e, c0 + 2])
                xj = plsc.load_gather(rj, [e, c0])
                yj = plsc.load_gather(rj, [e, c0 + 1])
                zj = plsc.load_gather(rj, [e, c0 + 2])
                dx = minimg(xi - xj, ibx, lxv)
                dy = minimg(yi - yj, iby, lyv)
                dz = minimg(zi - zj, ibz, lzv)
                r2 = dx * dx + dy * dy + dz * dz
                r2 = jnp.maximum(r2, 1e-6)
                s2 = sg * sg / r2
                tmp = s2 * s2 * s2
                ene = 4.0 * ep * tmp * (tmp - 1.0)
                ene = jnp.where(r2 < cut2, ene, jnp.zeros_like(ene))
                return acc + ene

            return lax.fori_loop(0, n_steps, step, acc)

        fetch(0, ibi_a, ibj_a, ri_a, rj_a, sem_a)

        def chunk_pair(t, acc):
            fetch(2 * t + 1, ibi_b, ibj_b, ri_b, rj_b, sem_b)
            drain(ibi_a, ibj_a, ri_a, rj_a, sem_a)
            acc = compute(ibi_a, ibj_a, ri_a, rj_a, acc)

            @pl.when(t + 1 < n_chunks // 2)
            def _():
                fetch(2 * t + 2, ibi_a, ibj_a, ri_a, rj_a, sem_a)

            drain(ibi_b, ibj_b, ri_b, rj_b, sem_b)
            return compute(ibi_b, ibj_b, ri_b, rj_b, acc)

        acc = lax.fori_loop(0, n_chunks // 2, chunk_pair,
                            jnp.zeros((_L,), jnp.float32))
        acc_v[...] = acc
        pltpu.sync_copy(acc_v, out.at[wid])

    return body


def _tc_sum(x_ref, o_ref):
    o_ref[0, 0] = jnp.sum(x_ref[...])


def kernel(coords, pairs, box, sigma, epsilon, cutoff, atom_types):
    n_pairs = pairs.shape[0]
    n_atoms = coords.shape[0]
    chunk = 2000
    assert n_pairs % (_NW * chunk) == 0 and (n_pairs // (_NW * chunk)) % 2 == 0

    coords8 = jnp.pad(coords.astype(jnp.float32), ((0, 0), (0, 5)))
    pi = pairs[:, 0].astype(jnp.int32)
    pj = pairs[:, 1].astype(jnp.int32)
    sig = sigma.astype(jnp.float32).reshape(-1)
    eps = epsilon.astype(jnp.float32).reshape(-1)
    box = box.astype(jnp.float32)
    inv_box = jnp.linalg.inv(box)
    cut = jnp.asarray(cutoff, jnp.float32)

    def bf(x):
        return x.astype(jnp.bfloat16).astype(jnp.float32)

    cst = jnp.concatenate([
        jnp.full((16,), bf(inv_box[0, 0]), jnp.float32),
        jnp.full((16,), bf(inv_box[1, 1]), jnp.float32),
        jnp.full((16,), bf(inv_box[2, 2]), jnp.float32),
        jnp.full((16,), bf(box[0, 0]), jnp.float32),
        jnp.full((16,), bf(box[1, 1]), jnp.float32),
        jnp.full((16,), bf(box[2, 2]), jnp.float32),
        jnp.full((16,), cut * cut, jnp.float32),
    ])

    parts = _sc_energy(n_pairs, n_atoms, chunk)(
        coords8, pi, pj, sig, eps, atom_types.astype(jnp.int32), cst)

    total = pl.pallas_call(
        _tc_sum,
        out_shape=jax.ShapeDtypeStruct((1, 1), jnp.float32),
        out_specs=pl.BlockSpec(memory_space=pltpu.SMEM),
    )(parts)
    return total[0, 0]


# speedup vs baseline: 12.8910x; 1.0833x over previous
import functools

import jax
import jax.numpy as jnp
from jax import lax
from jax.experimental import pallas as pl
from jax.experimental.pallas import tpu as pltpu
from jax.experimental.pallas import tpu_sc as plsc

_NC = 2
_NS = 16
_NW = _NC * _NS
_L = 16


def _sc_energy(n_pairs, n_atoms, chunk):
    n_per_w = n_pairs // _NW
    n_chunks = n_per_w // chunk
    n_steps = chunk // _L
    c2 = 2 * chunk

    mesh = plsc.VectorSubcoreMesh(core_axis_name="c", subcore_axis_name="s")

    @functools.partial(
        pl.kernel,
        mesh=mesh,
        compiler_params=pltpu.CompilerParams(
            needs_layout_passes=False, use_tc_tiling_on_sc=False),
        out_type=jax.ShapeDtypeStruct((_NW, _L), jnp.float32),
        scratch_types=[
            pltpu.VMEM((1024,), jnp.float32),
            pltpu.VMEM((1024,), jnp.float32),
            pltpu.VMEM((112,), jnp.float32),
            pltpu.VMEM((chunk,), jnp.int32),
            pltpu.VMEM((chunk,), jnp.int32),
            pltpu.VMEM((chunk,), jnp.int32),
            pltpu.VMEM((chunk,), jnp.int32),
            pltpu.VMEM((chunk, 8), jnp.float32),
            pltpu.VMEM((chunk, 8), jnp.float32),
            pltpu.VMEM((chunk, 8), jnp.float32),
            pltpu.VMEM((chunk, 8), jnp.float32),
            pltpu.VMEM((_L,), jnp.float32),
            pltpu.SemaphoreType.DMA,
            pltpu.SemaphoreType.DMA,
        ],
    )
    def body(coords8, pi, pj, sig, eps, cst, out,
             sig_v, eps_v, cst_v,
             ibi_a, ibj_a, ibi_b, ibj_b, ri_a, rj_a, ri_b, rj_b, acc_v,
             sem_a, sem_b):
        wid = lax.axis_index("s") * _NC + lax.axis_index("c")
        base_w = wid * n_per_w

        pltpu.sync_copy(sig, sig_v)
        pltpu.sync_copy(eps, eps_v)
        pltpu.sync_copy(cst, cst_v)

        ibx = cst_v[pl.ds(0, _L)]
        iby = cst_v[pl.ds(16, _L)]
        ibz = cst_v[pl.ds(32, _L)]
        lxv = cst_v[pl.ds(48, _L)]
        lyv = cst_v[pl.ds(64, _L)]
        lzv = cst_v[pl.ds(80, _L)]
        cut2 = cst_v[pl.ds(96, _L)]
        lanes = lax.iota(jnp.int32, _L)
        lanes2 = lanes * 2

        def bf16r(x):
            u = plsc.bitcast(x, jnp.int32)
            u = u + 0x7FFF + ((u >> 16) & 1)
            u = u & jnp.int32(-65536)
            return plsc.bitcast(u, jnp.float32)

        def minimg(d, ib, lv):
            s = bf16r(d) * ib
            f = jnp.where(s > 0.5, 1.0, 0.0) + jnp.where(s < -0.5, -1.0, 0.0)
            return bf16r(s - f) * lv

        def fetch(g, ibi, ibj, ri, rj, sem):
            base = pl.multiple_of(base_w + g * chunk, 8)
            pltpu.sync_copy(pi.at[pl.ds(base, chunk)], ibi)
            pltpu.sync_copy(pj.at[pl.ds(base, chunk)], ibj)
            pltpu.async_copy(coords8.at[ibi], ri, sem)
            pltpu.async_copy(coords8.at[ibj], rj, sem)

        def drain(ibi, ibj, ri, rj, sem):
            pltpu.make_async_copy(coords8.at[ibi], ri, sem).wait()
            pltpu.make_async_copy(coords8.at[ibj], rj, sem).wait()

        def compute(ibi, ibj, ri, rj, acc):
            def step(i, acc):
                o = i * _L
                e = lanes + o
                c0 = lanes - lanes
                ti = plsc.bitcast(plsc.load_gather(ri, [e, c0 + 3]),
                                  jnp.int32)
                tj = plsc.bitcast(plsc.load_gather(rj, [e, c0 + 3]),
                                  jnp.int32)
                tp = ti * 32 + tj
                sg = plsc.load_gather(sig_v, [tp])
                ep = plsc.load_gather(eps_v, [tp])
                xi = plsc.load_gather(ri, [e, c0])
                yi = plsc.load_gather(ri, [e, c0 + 1])
                zi = plsc.load_gather(ri, [e, c0 + 2])
                xj = plsc.load_gather(rj, [e, c0])
                yj = plsc.load_gather(rj, [e, c0 + 1])
                zj = plsc.load_gather(rj, [e, c0 + 2])
                dx = minimg(xi - xj, ibx, lxv)
                dy = minimg(yi - yj, iby, lyv)
                dz = minimg(zi - zj, ibz, lzv)
                r2 = dx * dx + dy * dy + dz * dz
                r2 = jnp.maximum(r2, 1e-6)
                s2 = sg * sg / r2
                tmp = s2 * s2 * s2
                ene = 4.0 * ep * tmp * (tmp - 1.0)
                ene = jnp.where(r2 < cut2, ene, jnp.zeros_like(ene))
                return acc + ene

            return lax.fori_loop(0, n_steps, step, acc)

        fetch(0, ibi_a, ibj_a, ri_a, rj_a, sem_a)

        def chunk_pair(t, acc):
            fetch(2 * t + 1, ibi_b, ibj_b, ri_b, rj_b, sem_b)
            drain(ibi_a, ibj_a, ri_a, rj_a, sem_a)
            acc = compute(ibi_a, ibj_a, ri_a, rj_a, acc)

            @pl.when(t + 1 < n_chunks // 2)
            def _():
                fetch(2 * t + 2, ibi_a, ibj_a, ri_a, rj_a, sem_a)

            drain(ibi_b, ibj_b, ri_b, rj_b, sem_b)
            return compute(ibi_b, ibj_b, ri_b, rj_b, acc)

        acc = lax.fori_loop(0, n_chunks // 2, chunk_pair,
                            jnp.zeros((_L,), jnp.float32))
        acc_v[...] = acc
        pltpu.sync_copy(acc_v, out.at[wid])

    return body


def _tc_sum(x_ref, o_ref):
    o_ref[0, 0] = jnp.sum(x_ref[...])


def kernel(coords, pairs, box, sigma, epsilon, cutoff, atom_types):
    n_pairs = pairs.shape[0]
    n_atoms = coords.shape[0]
    chunk = 2000
    assert n_pairs % (_NW * chunk) == 0 and (n_pairs // (_NW * chunk)) % 2 == 0

    tv = jax.lax.bitcast_convert_type(atom_types.astype(jnp.int32),
                                      jnp.float32)[:, None]
    coords8 = jnp.concatenate(
        [coords.astype(jnp.float32), tv,
         jnp.zeros((n_atoms, 4), jnp.float32)], axis=1)
    pi = pairs[:, 0].astype(jnp.int32)
    pj = pairs[:, 1].astype(jnp.int32)
    sig = sigma.astype(jnp.float32).reshape(-1)
    eps = epsilon.astype(jnp.float32).reshape(-1)
    box = box.astype(jnp.float32)
    bd = jnp.diagonal(box)
    ibd = 1.0 / bd
    cut = jnp.asarray(cutoff, jnp.float32)

    def bf(x):
        return x.astype(jnp.bfloat16).astype(jnp.float32)

    vals = jnp.concatenate([bf(ibd), bf(bd), (cut * cut)[None]])
    cst = jnp.repeat(vals, 16)

    parts = _sc_energy(n_pairs, n_atoms, chunk)(
        coords8, pi, pj, sig, eps, cst)

    total = pl.pallas_call(
        _tc_sum,
        out_shape=jax.ShapeDtypeStruct((1, 1), jnp.float32),
        out_specs=pl.BlockSpec(memory_space=pltpu.SMEM),
    )(parts)
    return total[0, 0]
